# baseline (speedup 1.0000x reference)
import jax
import jax.numpy as jnp
from jax.experimental import pallas as pl
from jax.experimental.pallas import tpu as pltpu

_SUB = 16
_LANES = 128
_CHUNK = _SUB * _LANES
_NSPLIT = 4


def _round_up(n, m):
    return ((n + m - 1) // m) * m


def _tree_sum(terms):
    while len(terms) > 1:
        nxt = [terms[i] + terms[i + 1] for i in range(0, len(terms) - 1, 2)]
        if len(terms) % 2:
            nxt.append(terms[-1])
        terms = nxt
    return terms[0]


def _mlp_chunk(x_ref, w, o_ref, s_in, s_out):
    t = x_ref[pl.ds(s_in, 2 * _SUB), :]
    a = t.reshape(_SUB, 2, _LANES)
    x0 = a[:, 0, :].astype(jnp.bfloat16)
    x1 = a[:, 1, :].astype(jnp.bfloat16)

    w1 = w[0:20]
    b1 = w[20:30]
    w2 = w[30:130]
    b2 = w[130:140]
    w3 = w[140:150]
    b3 = w[150]

    h1 = [jnp.maximum(w1[2 * j] * x0 + (w1[2 * j + 1] * x1 + b1[j]),
                      jnp.bfloat16(0))
          for j in range(10)]

    h2 = []
    for j in range(10):
        prods = [w2[j * 10 + k] * h1[k] for k in range(10)]
        prods.append(b2[j])
        h2.append(jnp.maximum(_tree_sum(prods), jnp.bfloat16(0)))

    prods = [w3[k] * h2[k] for k in range(10)]
    prods.append(b3)

    z = _tree_sum(prods).astype(jnp.float32)
    o_ref[pl.ds(s_out, _SUB), :] = 0.5 * (jnp.tanh(0.5 * z) + 1.0)


def _mlp_kernel(x0_ref, x1_ref, x2_ref, x3_ref, wf_ref, o_ref):
    w = [wf_ref[j] for j in range(151)]
    x_refs = (x0_ref, x1_ref, x2_ref, x3_ref)
    nq = o_ref.shape[0] // (_SUB * _NSPLIT)
    for q in range(_NSPLIT):
        for c in range(nq):
            _mlp_chunk(x_refs[q], w, o_ref,
                       c * 2 * _SUB, (q * nq + c) * _SUB)


def kernel(x, w1, b1, w2, b2, w3, b3):
    B = x.shape[0]
    tile_b = min(131072, _round_up(pl.cdiv(B, 8), _CHUNK * _NSPLIT))
    tile_b = max(_CHUNK * _NSPLIT, _round_up(tile_b, _CHUNK * _NSPLIT))
    Bp = _round_up(B, tile_b)
    n_tiles = Bp // tile_b
    c_tile = tile_b // _LANES

    xp = jnp.pad(x, ((0, Bp - B), (0, 0))) if Bp != B else x
    xv = (xp.reshape(Bp // _LANES, _LANES, 2)
          .transpose(0, 2, 1)
          .reshape(2 * (Bp // _LANES), _LANES))

    def splat(a):
        flat = a.reshape(-1).astype(jnp.bfloat16)
        return jnp.broadcast_to(flat[:, None, None],
                                (flat.shape[0], _SUB, _LANES))

    wf = jnp.concatenate([
        splat(w1), splat(b1), splat(w2), splat(b2), splat(w3), splat(b3),
    ], axis=0)

    qrows = 2 * c_tile // _NSPLIT

    def quarter_spec(k):
        return pl.BlockSpec((qrows, _LANES),
                            lambda i, k=k: (i * _NSPLIT + k, 0))

    out = pl.pallas_call(
        _mlp_kernel,
        out_shape=jax.ShapeDtypeStruct((Bp // _LANES, _LANES), jnp.float32),
        grid=(n_tiles,),
        in_specs=[
            quarter_spec(0), quarter_spec(1), quarter_spec(2), quarter_spec(3),
            pl.BlockSpec((151, _SUB, _LANES), lambda i: (0, 0, 0)),
        ],
        out_specs=pl.BlockSpec((c_tile, _LANES), lambda i: (i, 0)),
        compiler_params=pltpu.CompilerParams(
            dimension_semantics=("parallel",),
        ),
    )(xv, xv, xv, xv, wf)

    return out.reshape(Bp)[:B].reshape(B, 1)
